# Initial kernel scaffold; baseline (speedup 1.0000x reference)
import functools

import jax
import jax.numpy as jnp
from jax import lax
from jax.experimental import pallas as pl
from jax.experimental.pallas import tpu as pltpu
from jax.experimental.pallas import tpu_sc as plsc

N_USERS = 10000
N_ITEMS = 40000
N_NODES = N_USERS + N_ITEMS
D = 64
N_LAYERS = 3
N_EDGES = 800000

NUM_SC = 2
NUM_TILES = 16
K = 128
CHUNKS = (N_EDGES + NUM_TILES * K - 1) // (NUM_TILES * K)
E_PAD = NUM_TILES * CHUNKS * K
HALF = 25024
NP = NUM_SC * HALF
ROWS_PER_TILE = HALF // NUM_TILES

_mesh = plsc.VectorSubcoreMesh(core_axis_name="c", subcore_axis_name="s")


@functools.partial(
    pl.kernel,
    out_type=jax.ShapeDtypeStruct((NP, D), jnp.float32),
    mesh=_mesh,
    scratch_types=[
        pltpu.VMEM((K,), jnp.int32),
        pltpu.VMEM((K,), jnp.int32),
        pltpu.VMEM((K,), jnp.int32),
        pltpu.VMEM((K,), jnp.float32),
        pltpu.VMEM((K, D), jnp.float32),
        pltpu.VMEM_SHARED((HALF, D), jnp.float32),
        pltpu.SemaphoreType.DMA,
    ],
)
def _layer(table, src, dst, w, out, idx_v, dst_v, loc_v, w_v, rows, acc, sem):
    c = lax.axis_index("c")
    s = lax.axis_index("s")
    zero16 = jnp.zeros((16,), jnp.float32)

    def _zero_rows(j, _):
        for q in range(D // 16):
            rows[j, pl.ds(16 * q, 16)] = zero16
        return 0

    lax.fori_loop(0, K, _zero_rows, 0)
    acc_base = s * ROWS_PER_TILE
    n_full = ROWS_PER_TILE // K
    rem = ROWS_PER_TILE - n_full * K

    def _zero_acc(j, _):
        pltpu.sync_copy(rows, acc.at[pl.ds(acc_base + j * K, K)])
        return 0

    lax.fori_loop(0, n_full, _zero_acc, 0)
    pltpu.sync_copy(rows.at[pl.ds(0, rem)],
                    acc.at[pl.ds(acc_base + n_full * K, rem)])
    plsc.subcore_barrier()

    node_base = c * HALF
    iota16 = lax.iota(jnp.int32, (16,))

    def _chunk(j, _):
        base = (s * CHUNKS + j) * K
        pltpu.sync_copy(src.at[pl.ds(base, K)], idx_v)
        pltpu.sync_copy(dst.at[pl.ds(base, K)], dst_v)
        pltpu.sync_copy(w.at[pl.ds(base, K)], w_v)
        pltpu.async_copy(table.at[idx_v], rows, sem).wait()
        for g in range(K // 16):
            sl = pl.ds(16 * g, 16)
            d_raw = dst_v[sl]
            loc = d_raw - node_base
            in_half = (loc >= 0) & (loc < HALF)
            spread = lax.rem(base + 16 * g + iota16, HALF)
            loc_v[sl] = jnp.where(in_half, loc, spread)
            w_v[sl] = jnp.where(in_half, w_v[sl], 0.0)
        for e in range(K):
            wb = plsc.load_gather(w_v, [jnp.full((16,), e, jnp.int32)])
            for q in range(D // 16):
                sl = pl.ds(16 * q, 16)
                rows[e, sl] = rows[e, sl] * wb
        pltpu.sync_copy(rows, acc.at[loc_v], add=True)
        return 0

    lax.fori_loop(0, CHUNKS, _chunk, 0)
    plsc.subcore_barrier()

    out_base = node_base + acc_base

    def _copy_out(j, _):
        pltpu.sync_copy(acc.at[pl.ds(acc_base + j * K, K)],
                        out.at[pl.ds(out_base + j * K, K)])
        return 0

    lax.fori_loop(0, n_full, _copy_out, 0)
    pltpu.sync_copy(acc.at[pl.ds(acc_base + n_full * K, rem)],
                    out.at[pl.ds(out_base + n_full * K, rem)])


def _mean_kernel(t0, t1, t2, t3, o):
    o[...] = (t0[...] + t1[...] + t2[...] + t3[...]) * 0.25


_N_BLOCKS = 8
_BLOCK = NP // _N_BLOCKS


def _mean4(t0, t1, t2, t3):
    spec = pl.BlockSpec((_BLOCK, D), lambda i: (i, 0))
    return pl.pallas_call(
        _mean_kernel,
        out_shape=jax.ShapeDtypeStruct((NP, D), jnp.float32),
        grid=(_N_BLOCKS,),
        in_specs=[spec] * 4,
        out_specs=spec,
    )(t0, t1, t2, t3)


def kernel(users_emb, items_emb, edge_index, edge_weight):
    table0 = jnp.concatenate(
        [users_emb, items_emb,
         jnp.zeros((NP - N_NODES, D), jnp.float32)], axis=0)
    pad_e = E_PAD - N_EDGES
    src = jnp.concatenate(
        [edge_index[0].astype(jnp.int32),
         jnp.arange(pad_e, dtype=jnp.int32) % N_NODES])
    dst = jnp.concatenate(
        [edge_index[1].astype(jnp.int32), jnp.zeros((pad_e,), jnp.int32)])
    w = jnp.concatenate(
        [edge_weight, jnp.zeros((pad_e,), jnp.float32)])

    tables = [table0]
    for _ in range(N_LAYERS):
        tables.append(_layer(tables[-1], src, dst, w))
    light_out = _mean4(*tables)
    return (light_out[:N_USERS], light_out[N_USERS:N_NODES])


# speedup vs baseline: 3.0241x; 3.0241x over previous
import functools

import jax
import jax.numpy as jnp
from jax import lax
from jax.experimental import pallas as pl
from jax.experimental.pallas import tpu as pltpu
from jax.experimental.pallas import tpu_sc as plsc

_GDN = lax.GatherDimensionNumbers(
    offset_dims=(), collapsed_slice_dims=(0,), start_index_map=(0,))


def _lane_bcast(vec16, lane):
    idx = jnp.full((16, 1), lane, jnp.int32)
    return lax.gather(vec16, idx, _GDN, slice_sizes=(1,),
                      mode=lax.GatherScatterMode.PROMISE_IN_BOUNDS)


N_USERS = 10000
N_ITEMS = 40000
N_NODES = N_USERS + N_ITEMS
D = 64
N_LAYERS = 3
N_EDGES = 800000

NUM_SC = 2
NUM_TILES = 16
K = 128
CHUNKS = (N_EDGES + NUM_TILES * K - 1) // (NUM_TILES * K)
E_PAD = NUM_TILES * CHUNKS * K
HALF = 25088
NP = NUM_SC * HALF
ROWS_PER_TILE = HALF // NUM_TILES

_mesh = plsc.VectorSubcoreMesh(core_axis_name="c", subcore_axis_name="s")


@functools.partial(
    pl.kernel,
    out_type=jax.ShapeDtypeStruct((NP, D), jnp.float32),
    mesh=_mesh,
    compiler_params=pltpu.CompilerParams(needs_layout_passes=False,
                                         use_tc_tiling_on_sc=False),
    scratch_types=[
        pltpu.VMEM((K,), jnp.int32),
        pltpu.VMEM((K,), jnp.int32),
        pltpu.VMEM((K,), jnp.int32),
        pltpu.VMEM((K,), jnp.float32),
        pltpu.VMEM((K, D), jnp.float32),
        pltpu.VMEM_SHARED((HALF, D), jnp.float32),
        pltpu.SemaphoreType.DMA,
    ],
)
def _layer(table, src, dst, w, out, idx_v, dst_v, loc_v, w_v, rows, acc, sem):
    c = lax.axis_index("c")
    s = lax.axis_index("s")
    zero16 = jnp.zeros((16,), jnp.float32)

    def _zero_rows(j, _):
        for q in range(D // 16):
            rows[j, pl.ds(16 * q, 16)] = zero16
        return 0

    lax.fori_loop(0, K, _zero_rows, 0)
    acc_base = pl.multiple_of(s * ROWS_PER_TILE, 8)
    n_full = ROWS_PER_TILE // K
    rem = ROWS_PER_TILE - n_full * K

    def _zero_acc(j, _):
        pltpu.sync_copy(rows, acc.at[pl.ds(pl.multiple_of(acc_base + j * K, 8), K)])
        return 0

    lax.fori_loop(0, n_full, _zero_acc, 0)
    pltpu.sync_copy(rows.at[pl.ds(0, rem)],
                    acc.at[pl.ds(acc_base + n_full * K, rem)])
    plsc.subcore_barrier()

    node_base = pl.multiple_of(c * HALF, 8)
    iota16 = lax.iota(jnp.int32, 16)

    def _chunk(j, _):
        base = pl.multiple_of((s * CHUNKS + j) * K, 8)
        pltpu.sync_copy(src.at[pl.ds(base, K)], idx_v)
        pltpu.sync_copy(dst.at[pl.ds(base, K)], dst_v)
        pltpu.sync_copy(w.at[pl.ds(base, K)], w_v)
        pltpu.async_copy(table.at[idx_v], rows, sem).wait()
        for g in range(K // 16):
            sl = pl.ds(16 * g, 16)
            d_raw = dst_v[sl]
            loc = d_raw - node_base
            in_half = (loc >= 0) & (loc < HALF)
            spread = lax.rem(base + 16 * g + iota16, HALF)
            loc_v[sl] = jnp.where(in_half, loc, spread)
            w_v[sl] = jnp.where(in_half, w_v[sl], 0.0)
        for g in range(K // 16):
            wreg = w_v[pl.ds(16 * g, 16)]
            for l in range(16):
                e = 16 * g + l
                wb = _lane_bcast(wreg, l)
                for q in range(D // 16):
                    sl = pl.ds(16 * q, 16)
                    rows[e, sl] = rows[e, sl] * wb
        pltpu.sync_copy(rows, acc.at[loc_v], add=True)
        return 0

    lax.fori_loop(0, CHUNKS, _chunk, 0)
    plsc.subcore_barrier()

    out_base = pl.multiple_of(node_base + acc_base, 8)

    def _copy_out(j, _):
        pltpu.sync_copy(acc.at[pl.ds(pl.multiple_of(acc_base + j * K, 8), K)],
                        out.at[pl.ds(pl.multiple_of(out_base + j * K, 8), K)])
        return 0

    lax.fori_loop(0, n_full, _copy_out, 0)
    pltpu.sync_copy(acc.at[pl.ds(acc_base + n_full * K, rem)],
                    out.at[pl.ds(out_base + n_full * K, rem)])


def _mean_kernel(t0, t1, t2, t3, o):
    o[...] = (t0[...] + t1[...] + t2[...] + t3[...]) * 0.25


_N_BLOCKS = 8
_BLOCK = NP // _N_BLOCKS


def _mean4(t0, t1, t2, t3):
    spec = pl.BlockSpec((_BLOCK, D), lambda i: (i, 0))
    return pl.pallas_call(
        _mean_kernel,
        out_shape=jax.ShapeDtypeStruct((NP, D), jnp.float32),
        grid=(_N_BLOCKS,),
        in_specs=[spec] * 4,
        out_specs=spec,
    )(t0, t1, t2, t3)


def kernel(users_emb, items_emb, edge_index, edge_weight):
    table0 = jnp.concatenate(
        [users_emb, items_emb,
         jnp.zeros((NP - N_NODES, D), jnp.float32)], axis=0)
    pad_e = E_PAD - N_EDGES
    src = jnp.concatenate(
        [edge_index[0].astype(jnp.int32),
         jnp.arange(pad_e, dtype=jnp.int32) % N_NODES])
    dst = jnp.concatenate(
        [edge_index[1].astype(jnp.int32), jnp.zeros((pad_e,), jnp.int32)])
    w = jnp.concatenate(
        [edge_weight, jnp.zeros((pad_e,), jnp.float32)])

    tables = [table0]
    for _ in range(N_LAYERS):
        tables.append(_layer(tables[-1], src, dst, w))
    light_out = _mean4(*tables)
    return (light_out[:N_USERS], light_out[N_USERS:N_NODES])
